# baseline (device time: 339830 ns/iter reference)
import jax
import jax.numpy as jnp
from jax import lax
from jax.experimental import pallas as pl
from jax.experimental.pallas import tpu as pltpu

N_DEV = 4

_CompilerParams = getattr(pltpu, "CompilerParams", None) or getattr(
    pltpu, "TPUCompilerParams"
)


def kernel(x, router_W, route_idx, expert_W):
    n_per, d = x.shape
    n_exp = router_W.shape[1]
    e_per = expert_W.shape[0]
    h = expert_W.shape[2]

    x16 = x.astype(jnp.bfloat16)
    rw16 = router_W.astype(jnp.bfloat16)
    w16 = expert_W.astype(jnp.bfloat16)

    T = 512
    n_tiles = n_per // T

    def body(
        x_ref, rw_ref, idx_ref, w_hbm, out_ref,
        x_rem, g_own, g_rem, acc2, rs_recv, w_buf,
        ag_sx, ag_rx, ag_sg, ag_rg, rs_ss, rs_rs, rs3_ss, rs3_rs, w_sems,
    ):
        my = lax.axis_index("i")
        right = (my + 1) % N_DEV
        left = (my - 1) % N_DEV

        scores = jnp.dot(
            x_ref[...], rw_ref[...], preferred_element_type=jnp.float32
        )
        col = lax.broadcasted_iota(jnp.int32, (n_per, n_exp), 1)
        mask = (col == idx_ref[:, 0:1]) | (col == idx_ref[:, 1:2])
        ms = jnp.where(mask, scores, -1e9)
        m = jnp.max(ms, axis=1, keepdims=True)
        ex = jnp.exp(ms - m)
        g_own[...] = (ex / jnp.sum(ex, axis=1, keepdims=True)).astype(
            jnp.bfloat16
        )

        barrier = pltpu.get_barrier_semaphore()
        for nbr in (left, right):
            pl.semaphore_signal(
                barrier, inc=1, device_id=(nbr,),
                device_id_type=pl.DeviceIdType.MESH,
            )
        pl.semaphore_wait(barrier, 2)

        def ag_pair(t):
            src_x = x_ref if t == 0 else x_rem.at[t - 1]
            src_g = g_own if t == 0 else g_rem.at[t - 1]
            rx = pltpu.make_async_remote_copy(
                src_ref=src_x, dst_ref=x_rem.at[t],
                send_sem=ag_sx.at[t], recv_sem=ag_rx.at[t],
                device_id=(right,), device_id_type=pl.DeviceIdType.MESH,
            )
            rg = pltpu.make_async_remote_copy(
                src_ref=src_g, dst_ref=g_rem.at[t],
                send_sem=ag_sg.at[t], recv_sem=ag_rg.at[t],
                device_id=(right,), device_id_type=pl.DeviceIdType.MESH,
            )
            return rx, rg

        onehot = (
            lax.broadcasted_iota(jnp.int32, (n_exp, e_per), 0)
            == lax.broadcasted_iota(jnp.int32, (n_exp, e_per), 1) + e_per * my
        ).astype(jnp.bfloat16)
        eiota = lax.broadcasted_iota(jnp.int32, (n_per, e_per), 1)

        def contrib(read_x_tile, gb, aslot):
            gl = jnp.dot(gb, onehot, preferred_element_type=jnp.float32)
            for t in range(n_tiles):
                acc2[aslot, pl.ds(t * T, T), :] = jnp.zeros(
                    (T, h), jnp.bfloat16
                )
            pltpu.make_async_copy(
                w_hbm.at[0], w_buf.at[0], w_sems.at[0]
            ).start()

            def ebody(e, carry):
                slot = lax.rem(e, 2)
                pltpu.make_async_copy(
                    w_hbm.at[e], w_buf.at[slot], w_sems.at[slot]
                ).wait()

                @pl.when(e + 1 < e_per)
                def _():
                    nslot = lax.rem(e + 1, 2)
                    pltpu.make_async_copy(
                        w_hbm.at[e + 1], w_buf.at[nslot], w_sems.at[nslot]
                    ).start()

                gcol = jnp.sum(
                    gl * (eiota == e).astype(jnp.float32),
                    axis=1, keepdims=True,
                )
                for t in range(n_tiles):
                    sl = pl.ds(t * T, T)
                    y = jnp.dot(
                        read_x_tile(sl), w_buf[slot],
                        preferred_element_type=jnp.float32,
                    )
                    acc2[aslot, sl, :] = (
                        acc2[aslot, sl, :].astype(jnp.float32)
                        + y * gcol[t * T : (t + 1) * T, :]
                    ).astype(jnp.bfloat16)
                return carry

            lax.fori_loop(0, e_per, ebody, 0)

        rx0, rg0 = ag_pair(0)
        rx0.start()
        rg0.start()
        ag_list = [rx0, rg0]

        contrib(lambda sl: x_ref[sl, :], g_own[...], 0)
        for t in range(n_tiles):
            sl = pl.ds(t * T, T)
            out_ref[sl, :] = acc2[0, sl, :]

        rs_list = []
        for s in range(N_DEV - 2):
            ag_list[2 * s].wait_recv()
            ag_list[2 * s + 1].wait_recv()
            rxn, rgn = ag_pair(s + 1)
            rxn.start()
            rgn.start()
            ag_list += [rxn, rgn]

            aslot = s % 2
            contrib(lambda sl, s=s: x_rem[s, sl, :], g_rem[s], aslot)

            if s > 0:
                rs_list[s - 1].wait_recv()
                for t in range(n_tiles):
                    sl = pl.ds(t * T, T)
                    acc2[aslot, sl, :] = (
                        acc2[aslot, sl, :].astype(jnp.float32)
                        + rs_recv[s - 1, sl, :].astype(jnp.float32)
                    ).astype(jnp.bfloat16)
            r = pltpu.make_async_remote_copy(
                src_ref=acc2.at[aslot], dst_ref=rs_recv.at[s],
                send_sem=rs_ss.at[s], recv_sem=rs_rs.at[s],
                device_id=(right,), device_id_type=pl.DeviceIdType.MESH,
            )
            r.start()
            rs_list.append(r)

        s2 = N_DEV - 2
        rs_list[0].wait_send()
        ag_list[2 * s2].wait_recv()
        ag_list[2 * s2 + 1].wait_recv()
        gl2 = jnp.dot(g_rem[s2], onehot, preferred_element_type=jnp.float32)
        tile_list = []
        for t in range(n_tiles):
            sl = pl.ds(t * T, T)
            acc2[0, sl, :] = jnp.zeros((T, h), jnp.bfloat16)
            pltpu.make_async_copy(
                w_hbm.at[0], w_buf.at[0], w_sems.at[0]
            ).start()

            def ebody2(e, carry):
                slot = lax.rem(e, 2)
                pltpu.make_async_copy(
                    w_hbm.at[e], w_buf.at[slot], w_sems.at[slot]
                ).wait()

                @pl.when(e + 1 < e_per)
                def _():
                    nslot = lax.rem(e + 1, 2)
                    pltpu.make_async_copy(
                        w_hbm.at[e + 1], w_buf.at[nslot], w_sems.at[nslot]
                    ).start()

                gcol = jnp.sum(
                    gl2 * (eiota == e).astype(jnp.float32),
                    axis=1, keepdims=True,
                )
                y = jnp.dot(
                    x_rem[s2, sl, :], w_buf[slot],
                    preferred_element_type=jnp.float32,
                )
                acc2[0, sl, :] = (
                    acc2[0, sl, :].astype(jnp.float32)
                    + y * gcol[t * T : (t + 1) * T, :]
                ).astype(jnp.bfloat16)
                return carry

            lax.fori_loop(0, e_per, ebody2, 0)

            if t == 0:
                rs_list[s2 - 1].wait_recv()
            acc2[0, sl, :] = (
                acc2[0, sl, :].astype(jnp.float32)
                + rs_recv[s2 - 1, sl, :].astype(jnp.float32)
            ).astype(jnp.bfloat16)
            tr = pltpu.make_async_remote_copy(
                src_ref=acc2.at[0, sl], dst_ref=rs_recv.at[s2, sl],
                send_sem=rs3_ss.at[t], recv_sem=rs3_rs.at[t],
                device_id=(right,), device_id_type=pl.DeviceIdType.MESH,
            )
            tr.start()
            tile_list.append(tr)

        for t in range(n_tiles):
            sl = pl.ds(t * T, T)
            tile_list[t].wait_recv()
            out_ref[sl, :] = (
                out_ref[sl, :].astype(jnp.float32)
                + rs_recv[s2, sl, :].astype(jnp.float32)
            ).astype(jnp.bfloat16)

        for rr in ag_list:
            rr.wait_send()
        rs_list[1].wait_send()
        for tr in tile_list:
            tr.wait_send()

    return pl.pallas_call(
        body,
        out_shape=jax.ShapeDtypeStruct((n_per, h), jnp.bfloat16),
        in_specs=[
            pl.BlockSpec(memory_space=pltpu.VMEM),
            pl.BlockSpec(memory_space=pltpu.VMEM),
            pl.BlockSpec(memory_space=pltpu.VMEM),
            pl.BlockSpec(memory_space=pltpu.HBM),
        ],
        out_specs=pl.BlockSpec(memory_space=pltpu.VMEM),
        scratch_shapes=[
            pltpu.VMEM((N_DEV - 1, n_per, d), jnp.bfloat16),
            pltpu.VMEM((n_per, n_exp), jnp.bfloat16),
            pltpu.VMEM((N_DEV - 1, n_per, n_exp), jnp.bfloat16),
            pltpu.VMEM((2, n_per, h), jnp.bfloat16),
            pltpu.VMEM((N_DEV - 1, n_per, h), jnp.bfloat16),
            pltpu.VMEM((2, d, h), jnp.bfloat16),
            pltpu.SemaphoreType.DMA((N_DEV - 1,)),
            pltpu.SemaphoreType.DMA((N_DEV - 1,)),
            pltpu.SemaphoreType.DMA((N_DEV - 1,)),
            pltpu.SemaphoreType.DMA((N_DEV - 1,)),
            pltpu.SemaphoreType.DMA((N_DEV - 1,)),
            pltpu.SemaphoreType.DMA((N_DEV - 1,)),
            pltpu.SemaphoreType.DMA((4,)),
            pltpu.SemaphoreType.DMA((4,)),
            pltpu.SemaphoreType.DMA((2,)),
        ],
        compiler_params=_CompilerParams(
            collective_id=0, vmem_limit_bytes=52 * 1024 * 1024
        ),
    )(x16, rw16, route_idx, w16)


# device time: 331970 ns/iter; 1.0237x vs baseline; 1.0237x over previous
import jax
import jax.numpy as jnp
from jax import lax
from jax.experimental import pallas as pl
from jax.experimental.pallas import tpu as pltpu

N_DEV = 4

_CompilerParams = getattr(pltpu, "CompilerParams", None) or getattr(
    pltpu, "TPUCompilerParams"
)


def kernel(x, router_W, route_idx, expert_W):
    n_per, d = x.shape
    n_exp = router_W.shape[1]
    e_per = expert_W.shape[0]
    h = expert_W.shape[2]

    x16 = x.astype(jnp.bfloat16)
    rw16 = router_W.astype(jnp.bfloat16)
    w16 = expert_W.astype(jnp.bfloat16)

    T = 512
    n_tiles = n_per // T

    def body(
        x_ref, rw_ref, idx_ref, w_hbm, out_ref,
        x_rem, g_own, g_rem, acc2, rs_recv, w_buf,
        ag_sx, ag_rx, ag_sg, ag_rg, rs_ss, rs_rs, rs3_ss, rs3_rs, w_sems,
    ):
        my = lax.axis_index("i")
        right = (my + 1) % N_DEV
        left = (my - 1) % N_DEV

        scores = jnp.dot(
            x_ref[...], rw_ref[...], preferred_element_type=jnp.float32
        )
        col = lax.broadcasted_iota(jnp.int32, (n_per, n_exp), 1)
        mask = (col == idx_ref[:, 0:1]) | (col == idx_ref[:, 1:2])
        ms = jnp.where(mask, scores, -1e9)
        m = jnp.max(ms, axis=1, keepdims=True)
        ex = jnp.exp(ms - m)
        g_own[...] = (ex / jnp.sum(ex, axis=1, keepdims=True)).astype(
            jnp.bfloat16
        )

        barrier = pltpu.get_barrier_semaphore()
        for nbr in (left, right):
            pl.semaphore_signal(
                barrier, inc=1, device_id=(nbr,),
                device_id_type=pl.DeviceIdType.MESH,
            )
        pl.semaphore_wait(barrier, 2)

        def ag_pair(t):
            src_x = x_ref if t == 0 else x_rem.at[t - 1]
            src_g = g_own if t == 0 else g_rem.at[t - 1]
            rx = pltpu.make_async_remote_copy(
                src_ref=src_x, dst_ref=x_rem.at[t],
                send_sem=ag_sx.at[t], recv_sem=ag_rx.at[t],
                device_id=(right,), device_id_type=pl.DeviceIdType.MESH,
            )
            rg = pltpu.make_async_remote_copy(
                src_ref=src_g, dst_ref=g_rem.at[t],
                send_sem=ag_sg.at[t], recv_sem=ag_rg.at[t],
                device_id=(right,), device_id_type=pl.DeviceIdType.MESH,
            )
            return rx, rg

        onehot = (
            lax.broadcasted_iota(jnp.int32, (n_exp, e_per), 0)
            == lax.broadcasted_iota(jnp.int32, (n_exp, e_per), 1) + e_per * my
        ).astype(jnp.bfloat16)
        eiota = lax.broadcasted_iota(jnp.int32, (n_per, e_per), 1)

        def contrib(read_x_tile, gb, aslot):
            gl = jnp.dot(gb, onehot, preferred_element_type=jnp.float32)
            for t in range(n_tiles):
                acc2[aslot, pl.ds(t * T, T), :] = jnp.zeros(
                    (T, h), jnp.bfloat16
                )
            pltpu.make_async_copy(
                w_hbm.at[0], w_buf.at[0], w_sems.at[0]
            ).start()

            def ebody(e, carry):
                slot = lax.rem(e, 2)
                pltpu.make_async_copy(
                    w_hbm.at[e], w_buf.at[slot], w_sems.at[slot]
                ).wait()

                @pl.when(e + 1 < e_per)
                def _():
                    nslot = lax.rem(e + 1, 2)
                    pltpu.make_async_copy(
                        w_hbm.at[e + 1], w_buf.at[nslot], w_sems.at[nslot]
                    ).start()

                gcol = jnp.sum(
                    gl * (eiota == e).astype(jnp.float32),
                    axis=1, keepdims=True,
                )
                for t in range(n_tiles):
                    sl = pl.ds(t * T, T)
                    y = jnp.dot(
                        read_x_tile(sl), w_buf[slot],
                        preferred_element_type=jnp.float32,
                    )
                    acc2[aslot, sl, :] = (
                        acc2[aslot, sl, :].astype(jnp.float32)
                        + y * gcol[t * T : (t + 1) * T, :]
                    ).astype(jnp.bfloat16)
                return carry

            lax.fori_loop(0, e_per, ebody, 0)

        rx0, rg0 = ag_pair(0)
        rx0.start()
        rg0.start()
        ag_list = [rx0, rg0]

        contrib(lambda sl: x_ref[sl, :], g_own[...], 0)
        for t in range(n_tiles):
            sl = pl.ds(t * T, T)
            out_ref[sl, :] = acc2[0, sl, :]

        rs_list = []
        for s in range(N_DEV - 2):
            ag_list[2 * s].wait_recv()
            ag_list[2 * s + 1].wait_recv()
            rxn, rgn = ag_pair(s + 1)
            rxn.start()
            rgn.start()
            ag_list += [rxn, rgn]

            aslot = s % 2
            contrib(lambda sl, s=s: x_rem[s, sl, :], g_rem[s], aslot)

            if s > 0:
                rs_list[s - 1].wait_recv()
                for t in range(n_tiles):
                    sl = pl.ds(t * T, T)
                    acc2[aslot, sl, :] = (
                        acc2[aslot, sl, :].astype(jnp.float32)
                        + rs_recv[s - 1, sl, :].astype(jnp.float32)
                    ).astype(jnp.bfloat16)
            r = pltpu.make_async_remote_copy(
                src_ref=acc2.at[aslot], dst_ref=rs_recv.at[s],
                send_sem=rs_ss.at[s], recv_sem=rs_rs.at[s],
                device_id=(right,), device_id_type=pl.DeviceIdType.MESH,
            )
            r.start()
            rs_list.append(r)

        s2 = N_DEV - 2
        rs_list[0].wait_send()
        ag_list[2 * s2].wait_recv()
        ag_list[2 * s2 + 1].wait_recv()
        gl2 = jnp.dot(g_rem[s2], onehot, preferred_element_type=jnp.float32)
        for t in range(n_tiles):
            acc2[0, pl.ds(t * T, T), :] = jnp.zeros((T, h), jnp.bfloat16)
        pltpu.make_async_copy(w_hbm.at[0], w_buf.at[0], w_sems.at[0]).start()

        def ebody2(e, carry):
            slot = lax.rem(e, 2)
            pltpu.make_async_copy(
                w_hbm.at[e], w_buf.at[slot], w_sems.at[slot]
            ).wait()
            nslot = lax.rem(e + 1, 2)
            pltpu.make_async_copy(
                w_hbm.at[e + 1], w_buf.at[nslot], w_sems.at[nslot]
            ).start()
            gcol = jnp.sum(
                gl2 * (eiota == e).astype(jnp.float32),
                axis=1, keepdims=True,
            )
            for t in range(n_tiles):
                sl = pl.ds(t * T, T)
                y = jnp.dot(
                    x_rem[s2, sl, :], w_buf[slot],
                    preferred_element_type=jnp.float32,
                )
                acc2[0, sl, :] = (
                    acc2[0, sl, :].astype(jnp.float32)
                    + y * gcol[t * T : (t + 1) * T, :]
                ).astype(jnp.bfloat16)
            return carry

        lax.fori_loop(0, e_per - 1, ebody2, 0)

        elast = e_per - 1
        lslot = elast % 2
        pltpu.make_async_copy(
            w_hbm.at[elast], w_buf.at[lslot], w_sems.at[lslot]
        ).wait()
        rs_list[s2 - 1].wait_recv()
        tile_list = []
        for t in range(n_tiles):
            sl = pl.ds(t * T, T)
            y = jnp.dot(
                x_rem[s2, sl, :], w_buf[lslot],
                preferred_element_type=jnp.float32,
            )
            acc2[0, sl, :] = (
                acc2[0, sl, :].astype(jnp.float32)
                + y * gl2[t * T : (t + 1) * T, elast : elast + 1]
                + rs_recv[s2 - 1, sl, :].astype(jnp.float32)
            ).astype(jnp.bfloat16)
            tr = pltpu.make_async_remote_copy(
                src_ref=acc2.at[0, sl], dst_ref=rs_recv.at[s2, sl],
                send_sem=rs3_ss.at[t], recv_sem=rs3_rs.at[t],
                device_id=(right,), device_id_type=pl.DeviceIdType.MESH,
            )
            tr.start()
            tile_list.append(tr)

        for t in range(n_tiles):
            sl = pl.ds(t * T, T)
            tile_list[t].wait_recv()
            out_ref[sl, :] = (
                out_ref[sl, :].astype(jnp.float32)
                + rs_recv[s2, sl, :].astype(jnp.float32)
            ).astype(jnp.bfloat16)

        for rr in ag_list:
            rr.wait_send()
        rs_list[1].wait_send()
        for tr in tile_list:
            tr.wait_send()

    return pl.pallas_call(
        body,
        out_shape=jax.ShapeDtypeStruct((n_per, h), jnp.bfloat16),
        in_specs=[
            pl.BlockSpec(memory_space=pltpu.VMEM),
            pl.BlockSpec(memory_space=pltpu.VMEM),
            pl.BlockSpec(memory_space=pltpu.VMEM),
            pl.BlockSpec(memory_space=pltpu.HBM),
        ],
        out_specs=pl.BlockSpec(memory_space=pltpu.VMEM),
        scratch_shapes=[
            pltpu.VMEM((N_DEV - 1, n_per, d), jnp.bfloat16),
            pltpu.VMEM((n_per, n_exp), jnp.bfloat16),
            pltpu.VMEM((N_DEV - 1, n_per, n_exp), jnp.bfloat16),
            pltpu.VMEM((2, n_per, h), jnp.bfloat16),
            pltpu.VMEM((N_DEV - 1, n_per, h), jnp.bfloat16),
            pltpu.VMEM((2, d, h), jnp.bfloat16),
            pltpu.SemaphoreType.DMA((N_DEV - 1,)),
            pltpu.SemaphoreType.DMA((N_DEV - 1,)),
            pltpu.SemaphoreType.DMA((N_DEV - 1,)),
            pltpu.SemaphoreType.DMA((N_DEV - 1,)),
            pltpu.SemaphoreType.DMA((N_DEV - 1,)),
            pltpu.SemaphoreType.DMA((N_DEV - 1,)),
            pltpu.SemaphoreType.DMA((4,)),
            pltpu.SemaphoreType.DMA((4,)),
            pltpu.SemaphoreType.DMA((2,)),
        ],
        compiler_params=_CompilerParams(
            collective_id=0, vmem_limit_bytes=52 * 1024 * 1024
        ),
    )(x16, rw16, route_idx, w16)
